# baseline (speedup 1.0000x reference)
import jax
import jax.numpy as jnp
from jax.experimental import pallas as pl
from jax.experimental.pallas import tpu as pltpu

EMB_D = 64
K = 512
B_TOTAL = 256


def _vq_kernel(z_hbm, e_ref, o_hbm, z_buf, o_buf, sem_in, sem_out):
    cp_in = pltpu.make_async_copy(z_hbm, z_buf, sem_in)
    cp_in.start()
    e = e_ref[...]
    es = e * -2.0
    e2 = jnp.sum(e * e, axis=1, keepdims=True)
    e_hi = e.astype(jnp.bfloat16)
    e_lo = (e - e_hi.astype(jnp.float32)).astype(jnp.bfloat16)
    cp_in.wait()

    def body(g, carry):
        z = z_buf[g]
        d = e2 + jax.lax.dot_general(
            es, z, (((1,), (0,)), ((), ())),
            preferred_element_type=jnp.float32,
        )
        m = jnp.min(d, axis=0, keepdims=True)
        onehot = (d == m).astype(jnp.bfloat16)
        o_buf[g] = jax.lax.dot_general(
            e_hi, onehot, (((0,), (0,)), ((), ())),
            preferred_element_type=jnp.float32,
        ) + jax.lax.dot_general(
            e_lo, onehot, (((0,), (0,)), ((), ())),
            preferred_element_type=jnp.float32,
        )
        return carry

    jax.lax.fori_loop(0, B_TOTAL, body, 0)

    cp_out = pltpu.make_async_copy(o_buf, o_hbm, sem_out)
    cp_out.start()
    cp_out.wait()


@jax.jit
def kernel(z_e, e):
    B, C, H, W = z_e.shape
    HW = H * W
    z = z_e.reshape(B, C, HW)
    out = pl.pallas_call(
        _vq_kernel,
        in_specs=[
            pl.BlockSpec(memory_space=pltpu.MemorySpace.HBM),
            pl.BlockSpec(memory_space=pltpu.MemorySpace.VMEM),
        ],
        out_specs=pl.BlockSpec(memory_space=pltpu.MemorySpace.HBM),
        out_shape=jax.ShapeDtypeStruct((B, C, HW), jnp.float32),
        scratch_shapes=[
            pltpu.VMEM((B, C, HW), jnp.float32),
            pltpu.VMEM((B, C, HW), jnp.float32),
            pltpu.SemaphoreType.DMA,
            pltpu.SemaphoreType.DMA,
        ],
    )(z, e)
    return out.reshape(B, C, H, W)


# speedup vs baseline: 2.2123x; 2.2123x over previous
import jax
import jax.numpy as jnp
from jax.experimental import pallas as pl
from jax.experimental.pallas import tpu as pltpu

EMB_D = 64
K = 512
G = 128


def _vq_kernel(z_ref, e_ref, o_ref):
    e = e_ref[...]
    es = e * -2.0
    e2 = jnp.sum(e * e, axis=1, keepdims=True)
    e_hi = e.astype(jnp.bfloat16)
    e_lo = (e - e_hi.astype(jnp.float32)).astype(jnp.bfloat16)
    for g in range(G):
        z = z_ref[g]
        d = e2 + jax.lax.dot_general(
            es, z, (((1,), (0,)), ((), ())),
            preferred_element_type=jnp.float32,
        )
        m = jnp.min(d, axis=0, keepdims=True)
        onehot = (d == m).astype(jnp.bfloat16)
        o_ref[g] = jax.lax.dot_general(
            e_hi, onehot, (((0,), (0,)), ((), ())),
            preferred_element_type=jnp.float32,
        ) + jax.lax.dot_general(
            e_lo, onehot, (((0,), (0,)), ((), ())),
            preferred_element_type=jnp.float32,
        )


@jax.jit
def kernel(z_e, e):
    B, C, H, W = z_e.shape
    HW = H * W
    z = z_e.reshape(B, C, HW)
    out = pl.pallas_call(
        _vq_kernel,
        grid=(B // G,),
        in_specs=[
            pl.BlockSpec((G, C, HW), lambda i: (i, 0, 0)),
            pl.BlockSpec((K, EMB_D), lambda i: (0, 0)),
        ],
        out_specs=pl.BlockSpec((G, C, HW), lambda i: (i, 0, 0)),
        out_shape=jax.ShapeDtypeStruct((B, C, HW), jnp.float32),
        compiler_params=pltpu.CompilerParams(
            dimension_semantics=("parallel",),
        ),
    )(z, e)
    return out.reshape(B, C, H, W)


# speedup vs baseline: 2.2545x; 1.0191x over previous
import jax
import jax.numpy as jnp
from jax.experimental import pallas as pl
from jax.experimental.pallas import tpu as pltpu

EMB_D = 64
K = 512
G = 64


def _vq_kernel(z_ref, e_ref, o_ref):
    e = e_ref[...]
    es = e * -2.0
    e2 = jnp.sum(e * e, axis=1, keepdims=True)
    e_hi = e.astype(jnp.bfloat16)
    e_lo = (e - e_hi.astype(jnp.float32)).astype(jnp.bfloat16)
    for g in range(G):
        z = z_ref[g]
        d = e2 + jax.lax.dot_general(
            es, z, (((1,), (0,)), ((), ())),
            preferred_element_type=jnp.float32,
        )
        m = jnp.min(d, axis=0, keepdims=True)
        onehot = (d == m).astype(jnp.bfloat16)
        o_ref[g] = jax.lax.dot_general(
            e_hi, onehot, (((0,), (0,)), ((), ())),
            preferred_element_type=jnp.float32,
        ) + jax.lax.dot_general(
            e_lo, onehot, (((0,), (0,)), ((), ())),
            preferred_element_type=jnp.float32,
        )


@jax.jit
def kernel(z_e, e):
    B, C, H, W = z_e.shape
    HW = H * W
    z = z_e.reshape(B, C, HW)
    out = pl.pallas_call(
        _vq_kernel,
        grid=(B // G,),
        in_specs=[
            pl.BlockSpec((G, C, HW), lambda i: (i, 0, 0)),
            pl.BlockSpec((K, EMB_D), lambda i: (0, 0)),
        ],
        out_specs=pl.BlockSpec((G, C, HW), lambda i: (i, 0, 0)),
        out_shape=jax.ShapeDtypeStruct((B, C, HW), jnp.float32),
        compiler_params=pltpu.CompilerParams(
            dimension_semantics=("parallel",),
        ),
    )(z, e)
    return out.reshape(B, C, H, W)
